# baseline (device time: 445646 ns/iter reference)
import jax
import jax.numpy as jnp
from jax import lax
from jax.experimental import pallas as pl
from jax.experimental.pallas import tpu as pltpu

N_DEV = 8
DH = 64


def _neighbor_barrier(left, right):
    barrier = pltpu.get_barrier_semaphore()
    for nbr in (left, right):
        pl.semaphore_signal(
            barrier, inc=1, device_id=(nbr,), device_id_type=pl.DeviceIdType.MESH
        )
    pl.semaphore_wait(barrier, 2)


def _ring_allgather(x):
    B, s, D = x.shape

    def body(x_ref, out_ref, comm_ref, send_sems, recv_sems):
        my = lax.axis_index("i")
        left = lax.rem(my + N_DEV - 1, N_DEV)
        right = lax.rem(my + 1, N_DEV)
        _neighbor_barrier(left, right)

        out_ref[:, pl.ds(my * s, s), :] = x_ref[...]
        comm_ref[0] = x_ref[...]

        for h in range(N_DEV - 1):
            send_slot = h % 2
            recv_slot = (h + 1) % 2
            rdma = pltpu.make_async_remote_copy(
                src_ref=comm_ref.at[send_slot],
                dst_ref=comm_ref.at[recv_slot],
                send_sem=send_sems.at[send_slot],
                recv_sem=recv_sems.at[recv_slot],
                device_id=(right,),
                device_id_type=pl.DeviceIdType.MESH,
            )
            rdma.start()
            rdma.wait()
            origin = lax.rem(my + N_DEV - h - 1, N_DEV)
            out_ref[:, pl.ds(origin * s, s), :] = comm_ref[recv_slot]

    return pl.pallas_call(
        body,
        out_shape=jax.ShapeDtypeStruct((B, N_DEV * s, D), x.dtype),
        in_specs=[pl.BlockSpec(memory_space=pltpu.VMEM)],
        out_specs=pl.BlockSpec(memory_space=pltpu.VMEM),
        scratch_shapes=[
            pltpu.VMEM((2, B, s, D), x.dtype),
            pltpu.SemaphoreType.DMA((2,)),
            pltpu.SemaphoreType.DMA((2,)),
        ],
        compiler_params=pltpu.CompilerParams(collective_id=0),
    )(x)


def _ring_reduce_scatter(partial):
    B, Sf, D = partial.shape
    s = Sf // N_DEV

    def body(p_ref, out_ref, comm_ref, send_sems, recv_sems):
        my = lax.axis_index("i")
        left = lax.rem(my + N_DEV - 1, N_DEV)
        right = lax.rem(my + 1, N_DEV)
        _neighbor_barrier(left, right)

        c0 = lax.rem(my + N_DEV - 1, N_DEV)
        comm_ref[0] = p_ref[:, pl.ds(c0 * s, s), :]

        for st in range(N_DEV - 1):
            send_slot = st % 2
            recv_slot = (st + 1) % 2
            rdma = pltpu.make_async_remote_copy(
                src_ref=comm_ref.at[send_slot],
                dst_ref=comm_ref.at[recv_slot],
                send_sem=send_sems.at[send_slot],
                recv_sem=recv_sems.at[recv_slot],
                device_id=(right,),
                device_id_type=pl.DeviceIdType.MESH,
            )
            rdma.start()
            rdma.wait()
            c = lax.rem(my + 2 * N_DEV - st - 2, N_DEV)
            if st < N_DEV - 2:
                comm_ref[recv_slot] = comm_ref[recv_slot] + p_ref[:, pl.ds(c * s, s), :]
            else:
                out_ref[...] = comm_ref[recv_slot] + p_ref[:, pl.ds(c * s, s), :]

    return pl.pallas_call(
        body,
        out_shape=jax.ShapeDtypeStruct((B, s, D), partial.dtype),
        in_specs=[pl.BlockSpec(memory_space=pltpu.VMEM)],
        out_specs=pl.BlockSpec(memory_space=pltpu.VMEM),
        scratch_shapes=[
            pltpu.VMEM((2, B, s, D), partial.dtype),
            pltpu.SemaphoreType.DMA((2,)),
            pltpu.SemaphoreType.DMA((2,)),
        ],
        compiler_params=pltpu.CompilerParams(collective_id=1),
    )(partial)


def _rope(t, cos, sin):
    B, S, H, _ = t.shape
    t2 = t.reshape(B, S, H, DH // 2, 2)
    t_r = jnp.stack([-t2[..., 1], t2[..., 0]], axis=-1).reshape(B, S, H, DH)
    return t * cos[None, :, None, :] + t_r * sin[None, :, None, :]


def kernel(x, Wq, Wk, Wv, Wo):
    B, s, D = x.shape
    S = N_DEV * s
    H = Wq.shape[1] // DH

    xg = _ring_allgather(x)

    q = (xg.reshape(B * S, D) @ Wq).reshape(B, S, H, DH)
    k = (xg.reshape(B * S, D) @ Wk).reshape(B, S, H, DH)
    v = (xg.reshape(B * S, D) @ Wv).reshape(B, S, H, DH)

    inv = 1.0 / (10000.0 ** (jnp.arange(0, DH, 2, dtype=jnp.float32) / DH))
    pos = jnp.arange(S, dtype=jnp.float32)[:, None] * inv[None, :]
    cos = jnp.repeat(jnp.cos(pos), 2, axis=-1)
    sin = jnp.repeat(jnp.sin(pos), 2, axis=-1)
    q = _rope(q, cos, sin)
    k = _rope(k, cos, sin)

    scores = jnp.einsum("bihd,bjhd->bhij", q, k) * 0.125
    w = jax.nn.softmax(scores, axis=-1)
    ctx = jnp.einsum("bhij,bjhd->bihd", w, v).reshape(B, S, H * DH)
    partial = ctx @ Wo

    return _ring_reduce_scatter(partial)


# device time: 336797 ns/iter; 1.3232x vs baseline; 1.3232x over previous
import jax
import jax.numpy as jnp
from jax import lax
from jax.experimental import pallas as pl
from jax.experimental.pallas import tpu as pltpu

N_DEV = 8
DH = 64


def _neighbor_barrier(left, right):
    barrier = pltpu.get_barrier_semaphore()
    for nbr in (left, right):
        pl.semaphore_signal(
            barrier, inc=1, device_id=(nbr,), device_id_type=pl.DeviceIdType.MESH
        )
    pl.semaphore_wait(barrier, 2)


def _ring_allgather(x):
    B, s, D = x.shape
    R_HOPS = N_DEV // 2
    L_HOPS = N_DEV - 1 - R_HOPS

    def body(x_ref, out_ref, comm_r, comm_l, send_r, recv_r, send_l, recv_l):
        my = lax.axis_index("i")
        left = lax.rem(my + N_DEV - 1, N_DEV)
        right = lax.rem(my + 1, N_DEV)
        _neighbor_barrier(left, right)

        out_ref[:, pl.ds(my * s, s), :] = x_ref[...]
        comm_r[0] = x_ref[...]
        comm_l[0] = x_ref[...]

        for h in range(R_HOPS):
            ss, rs = h % 2, (h + 1) % 2
            rdma_r = pltpu.make_async_remote_copy(
                src_ref=comm_r.at[ss],
                dst_ref=comm_r.at[rs],
                send_sem=send_r.at[ss],
                recv_sem=recv_r.at[rs],
                device_id=(right,),
                device_id_type=pl.DeviceIdType.MESH,
            )
            rdma_r.start()
            if h < L_HOPS:
                rdma_l = pltpu.make_async_remote_copy(
                    src_ref=comm_l.at[ss],
                    dst_ref=comm_l.at[rs],
                    send_sem=send_l.at[ss],
                    recv_sem=recv_l.at[rs],
                    device_id=(left,),
                    device_id_type=pl.DeviceIdType.MESH,
                )
                rdma_l.start()
            rdma_r.wait()
            origin_r = lax.rem(my + N_DEV - h - 1, N_DEV)
            out_ref[:, pl.ds(origin_r * s, s), :] = comm_r[rs]
            if h < L_HOPS:
                rdma_l.wait()
                origin_l = lax.rem(my + h + 1, N_DEV)
                out_ref[:, pl.ds(origin_l * s, s), :] = comm_l[rs]

    return pl.pallas_call(
        body,
        out_shape=jax.ShapeDtypeStruct((B, N_DEV * s, D), x.dtype),
        in_specs=[pl.BlockSpec(memory_space=pltpu.VMEM)],
        out_specs=pl.BlockSpec(memory_space=pltpu.VMEM),
        scratch_shapes=[
            pltpu.VMEM((2, B, s, D), x.dtype),
            pltpu.VMEM((2, B, s, D), x.dtype),
            pltpu.SemaphoreType.DMA((2,)),
            pltpu.SemaphoreType.DMA((2,)),
            pltpu.SemaphoreType.DMA((2,)),
            pltpu.SemaphoreType.DMA((2,)),
        ],
        compiler_params=pltpu.CompilerParams(collective_id=0),
    )(x)


def _ring_reduce_scatter(partial):
    B, Sf, D = partial.shape
    s = Sf // N_DEV

    R_HOPS = N_DEV // 2
    L_HOPS = N_DEV - 1 - R_HOPS

    def body(p_ref, out_ref, comm_r, comm_l, send_r, recv_r, send_l, recv_l):
        my = lax.axis_index("i")
        left = lax.rem(my + N_DEV - 1, N_DEV)
        right = lax.rem(my + 1, N_DEV)
        _neighbor_barrier(left, right)

        c0r = lax.rem(my + R_HOPS, N_DEV)
        comm_r[0] = p_ref[:, pl.ds(c0r * s, s), :]
        c0l = lax.rem(my + N_DEV - L_HOPS, N_DEV)
        comm_l[0] = p_ref[:, pl.ds(c0l * s, s), :]

        for st in range(R_HOPS):
            ss, rs = st % 2, (st + 1) % 2
            rdma_r = pltpu.make_async_remote_copy(
                src_ref=comm_r.at[ss],
                dst_ref=comm_r.at[rs],
                send_sem=send_r.at[ss],
                recv_sem=recv_r.at[rs],
                device_id=(right,),
                device_id_type=pl.DeviceIdType.MESH,
            )
            rdma_r.start()
            if st < L_HOPS:
                rdma_l = pltpu.make_async_remote_copy(
                    src_ref=comm_l.at[ss],
                    dst_ref=comm_l.at[rs],
                    send_sem=send_l.at[ss],
                    recv_sem=recv_l.at[rs],
                    device_id=(left,),
                    device_id_type=pl.DeviceIdType.MESH,
                )
                rdma_l.start()
            rdma_r.wait()
            if st < R_HOPS - 1:
                cr = lax.rem(my + N_DEV + L_HOPS - st, N_DEV)
                comm_r[rs] = comm_r[rs] + p_ref[:, pl.ds(cr * s, s), :]
            if st < L_HOPS:
                rdma_l.wait()
                if st < L_HOPS - 1:
                    cl = lax.rem(my + N_DEV - 2 + st, N_DEV)
                    comm_l[rs] = comm_l[rs] + p_ref[:, pl.ds(cl * s, s), :]

        out_ref[...] = (
            p_ref[:, pl.ds(my * s, s), :]
            + comm_r[R_HOPS % 2]
            + comm_l[L_HOPS % 2]
        )

    return pl.pallas_call(
        body,
        out_shape=jax.ShapeDtypeStruct((B, s, D), partial.dtype),
        in_specs=[pl.BlockSpec(memory_space=pltpu.VMEM)],
        out_specs=pl.BlockSpec(memory_space=pltpu.VMEM),
        scratch_shapes=[
            pltpu.VMEM((2, B, s, D), partial.dtype),
            pltpu.VMEM((2, B, s, D), partial.dtype),
            pltpu.SemaphoreType.DMA((2,)),
            pltpu.SemaphoreType.DMA((2,)),
            pltpu.SemaphoreType.DMA((2,)),
            pltpu.SemaphoreType.DMA((2,)),
        ],
        compiler_params=pltpu.CompilerParams(collective_id=1),
    )(partial)


def _rope(t, cos, sin):
    B, S, H, _ = t.shape
    t2 = t.reshape(B, S, H, DH // 2, 2)
    t_r = jnp.stack([-t2[..., 1], t2[..., 0]], axis=-1).reshape(B, S, H, DH)
    return t * cos[None, :, None, :] + t_r * sin[None, :, None, :]


def kernel(x, Wq, Wk, Wv, Wo):
    B, s, D = x.shape
    S = N_DEV * s
    H = Wq.shape[1] // DH

    xg = _ring_allgather(x)

    q = (xg.reshape(B * S, D) @ Wq).reshape(B, S, H, DH)
    k = (xg.reshape(B * S, D) @ Wk).reshape(B, S, H, DH)
    v = (xg.reshape(B * S, D) @ Wv).reshape(B, S, H, DH)

    inv = 1.0 / (10000.0 ** (jnp.arange(0, DH, 2, dtype=jnp.float32) / DH))
    pos = jnp.arange(S, dtype=jnp.float32)[:, None] * inv[None, :]
    cos = jnp.repeat(jnp.cos(pos), 2, axis=-1)
    sin = jnp.repeat(jnp.sin(pos), 2, axis=-1)
    q = _rope(q, cos, sin)
    k = _rope(k, cos, sin)

    scores = jnp.einsum("bihd,bjhd->bhij", q, k) * 0.125
    w = jax.nn.softmax(scores, axis=-1)
    ctx = jnp.einsum("bhij,bjhd->bihd", w, v).reshape(B, S, H * DH)
    partial = ctx @ Wo

    return _ring_reduce_scatter(partial)


# device time: 296331 ns/iter; 1.5039x vs baseline; 1.1366x over previous
import jax
import jax.numpy as jnp
from jax import lax
from jax.experimental import pallas as pl
from jax.experimental.pallas import tpu as pltpu

N_DEV = 8
DH = 64


def _neighbor_barrier(left, right):
    barrier = pltpu.get_barrier_semaphore()
    for nbr in (left, right):
        pl.semaphore_signal(
            barrier, inc=1, device_id=(nbr,), device_id_type=pl.DeviceIdType.MESH
        )
    pl.semaphore_wait(barrier, 2)


def _ring_allgather(x):
    B, s, D = x.shape
    R_HOPS = N_DEV // 2
    L_HOPS = N_DEV - 1 - R_HOPS

    def body(x_ref, out_ref, comm_r, comm_l, send_r, recv_r, send_l, recv_l):
        my = lax.axis_index("i")
        left = lax.rem(my + N_DEV - 1, N_DEV)
        right = lax.rem(my + 1, N_DEV)
        _neighbor_barrier(left, right)

        out_ref[:, pl.ds(my * s, s), :] = x_ref[...]
        comm_r[0] = x_ref[...]
        comm_l[0] = x_ref[...]

        for h in range(R_HOPS):
            ss, rs = h % 2, (h + 1) % 2
            rdma_r = pltpu.make_async_remote_copy(
                src_ref=comm_r.at[ss],
                dst_ref=comm_r.at[rs],
                send_sem=send_r.at[ss],
                recv_sem=recv_r.at[rs],
                device_id=(right,),
                device_id_type=pl.DeviceIdType.MESH,
            )
            rdma_r.start()
            if h < L_HOPS:
                rdma_l = pltpu.make_async_remote_copy(
                    src_ref=comm_l.at[ss],
                    dst_ref=comm_l.at[rs],
                    send_sem=send_l.at[ss],
                    recv_sem=recv_l.at[rs],
                    device_id=(left,),
                    device_id_type=pl.DeviceIdType.MESH,
                )
                rdma_l.start()
            rdma_r.wait()
            origin_r = lax.rem(my + N_DEV - h - 1, N_DEV)
            out_ref[:, pl.ds(origin_r * s, s), :] = comm_r[rs]
            if h < L_HOPS:
                rdma_l.wait()
                origin_l = lax.rem(my + h + 1, N_DEV)
                out_ref[:, pl.ds(origin_l * s, s), :] = comm_l[rs]

    return pl.pallas_call(
        body,
        out_shape=jax.ShapeDtypeStruct((B, N_DEV * s, D), x.dtype),
        in_specs=[pl.BlockSpec(memory_space=pltpu.VMEM)],
        out_specs=pl.BlockSpec(memory_space=pltpu.VMEM),
        scratch_shapes=[
            pltpu.VMEM((2, B, s, D), x.dtype),
            pltpu.VMEM((2, B, s, D), x.dtype),
            pltpu.SemaphoreType.DMA((2,)),
            pltpu.SemaphoreType.DMA((2,)),
            pltpu.SemaphoreType.DMA((2,)),
            pltpu.SemaphoreType.DMA((2,)),
        ],
        compiler_params=pltpu.CompilerParams(collective_id=0),
    )(x)


def _ring_reduce_scatter(partial):
    B, Sf, D = partial.shape
    s = Sf // N_DEV

    R_HOPS = N_DEV // 2
    L_HOPS = N_DEV - 1 - R_HOPS

    def body(p_ref, out_ref, comm_r, comm_l, send_r, recv_r, send_l, recv_l):
        my = lax.axis_index("i")
        left = lax.rem(my + N_DEV - 1, N_DEV)
        right = lax.rem(my + 1, N_DEV)
        _neighbor_barrier(left, right)

        c0r = lax.rem(my + R_HOPS, N_DEV)
        comm_r[0] = p_ref[:, pl.ds(c0r * s, s), :]
        c0l = lax.rem(my + N_DEV - L_HOPS, N_DEV)
        comm_l[0] = p_ref[:, pl.ds(c0l * s, s), :]

        for st in range(R_HOPS):
            ss, rs = st % 2, (st + 1) % 2
            rdma_r = pltpu.make_async_remote_copy(
                src_ref=comm_r.at[ss],
                dst_ref=comm_r.at[rs],
                send_sem=send_r.at[ss],
                recv_sem=recv_r.at[rs],
                device_id=(right,),
                device_id_type=pl.DeviceIdType.MESH,
            )
            rdma_r.start()
            if st < L_HOPS:
                rdma_l = pltpu.make_async_remote_copy(
                    src_ref=comm_l.at[ss],
                    dst_ref=comm_l.at[rs],
                    send_sem=send_l.at[ss],
                    recv_sem=recv_l.at[rs],
                    device_id=(left,),
                    device_id_type=pl.DeviceIdType.MESH,
                )
                rdma_l.start()
            rdma_r.wait()
            if st < R_HOPS - 1:
                cr = lax.rem(my + N_DEV + L_HOPS - st, N_DEV)
                comm_r[rs] = comm_r[rs] + p_ref[:, pl.ds(cr * s, s), :]
            if st < L_HOPS:
                rdma_l.wait()
                if st < L_HOPS - 1:
                    cl = lax.rem(my + N_DEV - 2 + st, N_DEV)
                    comm_l[rs] = comm_l[rs] + p_ref[:, pl.ds(cl * s, s), :]

        out_ref[...] = (
            p_ref[:, pl.ds(my * s, s), :]
            + comm_r[R_HOPS % 2]
            + comm_l[L_HOPS % 2]
        )

    return pl.pallas_call(
        body,
        out_shape=jax.ShapeDtypeStruct((B, s, D), partial.dtype),
        in_specs=[pl.BlockSpec(memory_space=pltpu.VMEM)],
        out_specs=pl.BlockSpec(memory_space=pltpu.VMEM),
        scratch_shapes=[
            pltpu.VMEM((2, B, s, D), partial.dtype),
            pltpu.VMEM((2, B, s, D), partial.dtype),
            pltpu.SemaphoreType.DMA((2,)),
            pltpu.SemaphoreType.DMA((2,)),
            pltpu.SemaphoreType.DMA((2,)),
            pltpu.SemaphoreType.DMA((2,)),
        ],
        compiler_params=pltpu.CompilerParams(collective_id=1),
    )(partial)


def _attention(xg, Wq, Wk, Wv, Wo, cos, sin, P):
    B, S, D = xg.shape
    H = Wq.shape[1] // DH
    Wq_h = Wq.reshape(D, H, DH).transpose(1, 0, 2)
    Wk_h = Wk.reshape(D, H, DH).transpose(1, 0, 2)
    Wv_h = Wv.reshape(D, H, DH).transpose(1, 0, 2)
    Wo_h = Wo.reshape(H, DH, D)

    def body(x_ref, wq_ref, wk_ref, wv_ref, wo_ref, cos_ref, sin_ref, p_ref, out_ref):
        h = pl.program_id(1)
        x_b = x_ref[0]
        cos_v = cos_ref[...]
        sin_v = sin_ref[...]
        p_mat = p_ref[...]
        q = jnp.dot(x_b, wq_ref[0], preferred_element_type=jnp.float32)
        k = jnp.dot(x_b, wk_ref[0], preferred_element_type=jnp.float32)
        v = jnp.dot(x_b, wv_ref[0], preferred_element_type=jnp.float32)
        q = q * cos_v + jnp.dot(q, p_mat) * sin_v
        k = k * cos_v + jnp.dot(k, p_mat) * sin_v
        s_ = (
            lax.dot_general(
                q, k, (((1,), (1,)), ((), ())), preferred_element_type=jnp.float32
            )
            * 0.125
        )
        m = jnp.max(s_, axis=1, keepdims=True)
        e = jnp.exp(s_ - m)
        w = e / jnp.sum(e, axis=1, keepdims=True)
        ctx = jnp.dot(w, v, preferred_element_type=jnp.float32)
        contrib = jnp.dot(ctx, wo_ref[0], preferred_element_type=jnp.float32)

        @pl.when(h == 0)
        def _():
            out_ref[0] = contrib

        @pl.when(h != 0)
        def _():
            out_ref[0] = out_ref[0] + contrib

    return pl.pallas_call(
        body,
        grid=(B, H),
        in_specs=[
            pl.BlockSpec((1, S, D), lambda b, h: (b, 0, 0)),
            pl.BlockSpec((1, D, DH), lambda b, h: (h, 0, 0)),
            pl.BlockSpec((1, D, DH), lambda b, h: (h, 0, 0)),
            pl.BlockSpec((1, D, DH), lambda b, h: (h, 0, 0)),
            pl.BlockSpec((1, DH, D), lambda b, h: (h, 0, 0)),
            pl.BlockSpec((S, DH), lambda b, h: (0, 0)),
            pl.BlockSpec((S, DH), lambda b, h: (0, 0)),
            pl.BlockSpec((DH, DH), lambda b, h: (0, 0)),
        ],
        out_specs=pl.BlockSpec((1, S, D), lambda b, h: (b, 0, 0)),
        out_shape=jax.ShapeDtypeStruct((B, S, D), jnp.float32),
    )(xg, Wq_h, Wk_h, Wv_h, Wo_h, cos, sin, P)


def kernel(x, Wq, Wk, Wv, Wo):
    B, s, D = x.shape
    S = N_DEV * s

    xg = _ring_allgather(x)

    inv = 1.0 / (10000.0 ** (jnp.arange(0, DH, 2, dtype=jnp.float32) / DH))
    pos = jnp.arange(S, dtype=jnp.float32)[:, None] * inv[None, :]
    cos = jnp.repeat(jnp.cos(pos), 2, axis=-1)
    sin = jnp.repeat(jnp.sin(pos), 2, axis=-1)
    row = jnp.arange(DH)[:, None]
    col = jnp.arange(DH)[None, :]
    P = ((col == row + 1) & (row % 2 == 0)).astype(jnp.float32) - (
        (col == row - 1) & (row % 2 == 1)
    ).astype(jnp.float32)

    partial = _attention(xg, Wq, Wk, Wv, Wo, cos, sin, P)
    return _ring_reduce_scatter(partial)


# device time: 222130 ns/iter; 2.0062x vs baseline; 1.3340x over previous
import jax
import jax.numpy as jnp
from jax import lax
from jax.experimental import pallas as pl
from jax.experimental.pallas import tpu as pltpu

N_DEV = 8
DH = 64


def _ring2log(t):
    return jnp.where(t < 4, t, 11 - t)


def _neighbor_barrier(left, right):
    barrier = pltpu.get_barrier_semaphore()
    for nbr in (left, right):
        pl.semaphore_signal(
            barrier, inc=1, device_id=(nbr,), device_id_type=pl.DeviceIdType.MESH
        )
    pl.semaphore_wait(barrier, 2)


def _ring_allgather(x):
    B, s, D = x.shape
    R_HOPS = N_DEV // 2
    L_HOPS = N_DEV - 1 - R_HOPS

    def body(x_ref, out_ref, comm_r, comm_l, send_r, recv_r, send_l, recv_l):
        my = lax.axis_index("i")
        r = _ring2log(my)
        left = _ring2log(lax.rem(r + N_DEV - 1, N_DEV))
        right = _ring2log(lax.rem(r + 1, N_DEV))
        _neighbor_barrier(left, right)

        out_ref[:, pl.ds(my * s, s), :] = x_ref[...]
        comm_r[0] = x_ref[...]
        comm_l[0] = x_ref[...]

        for h in range(R_HOPS):
            ss, rs = h % 2, (h + 1) % 2
            rdma_r = pltpu.make_async_remote_copy(
                src_ref=comm_r.at[ss],
                dst_ref=comm_r.at[rs],
                send_sem=send_r.at[ss],
                recv_sem=recv_r.at[rs],
                device_id=(right,),
                device_id_type=pl.DeviceIdType.MESH,
            )
            rdma_r.start()
            if h < L_HOPS:
                rdma_l = pltpu.make_async_remote_copy(
                    src_ref=comm_l.at[ss],
                    dst_ref=comm_l.at[rs],
                    send_sem=send_l.at[ss],
                    recv_sem=recv_l.at[rs],
                    device_id=(left,),
                    device_id_type=pl.DeviceIdType.MESH,
                )
                rdma_l.start()
            rdma_r.wait()
            origin_r = _ring2log(lax.rem(r + N_DEV - h - 1, N_DEV))
            out_ref[:, pl.ds(origin_r * s, s), :] = comm_r[rs]
            if h < L_HOPS:
                rdma_l.wait()
                origin_l = _ring2log(lax.rem(r + h + 1, N_DEV))
                out_ref[:, pl.ds(origin_l * s, s), :] = comm_l[rs]

    return pl.pallas_call(
        body,
        out_shape=jax.ShapeDtypeStruct((B, N_DEV * s, D), x.dtype),
        in_specs=[pl.BlockSpec(memory_space=pltpu.VMEM)],
        out_specs=pl.BlockSpec(memory_space=pltpu.VMEM),
        scratch_shapes=[
            pltpu.VMEM((2, B, s, D), x.dtype),
            pltpu.VMEM((2, B, s, D), x.dtype),
            pltpu.SemaphoreType.DMA((2,)),
            pltpu.SemaphoreType.DMA((2,)),
            pltpu.SemaphoreType.DMA((2,)),
            pltpu.SemaphoreType.DMA((2,)),
        ],
        compiler_params=pltpu.CompilerParams(collective_id=0),
    )(x)


def _ring_reduce_scatter(partial):
    B, Sf, D = partial.shape
    s = Sf // N_DEV

    R_HOPS = N_DEV // 2
    L_HOPS = N_DEV - 1 - R_HOPS

    def body(p_ref, out_ref, comm_r, comm_l, send_r, recv_r, send_l, recv_l):
        my = lax.axis_index("i")
        r = _ring2log(my)
        left = _ring2log(lax.rem(r + N_DEV - 1, N_DEV))
        right = _ring2log(lax.rem(r + 1, N_DEV))
        _neighbor_barrier(left, right)

        c0r = _ring2log(lax.rem(r + R_HOPS, N_DEV))
        comm_r[0] = p_ref[:, pl.ds(c0r * s, s), :].astype(jnp.bfloat16)
        c0l = _ring2log(lax.rem(r + N_DEV - L_HOPS, N_DEV))
        comm_l[0] = p_ref[:, pl.ds(c0l * s, s), :].astype(jnp.bfloat16)

        for st in range(R_HOPS):
            ss, rs = st % 2, (st + 1) % 2
            rdma_r = pltpu.make_async_remote_copy(
                src_ref=comm_r.at[ss],
                dst_ref=comm_r.at[rs],
                send_sem=send_r.at[ss],
                recv_sem=recv_r.at[rs],
                device_id=(right,),
                device_id_type=pl.DeviceIdType.MESH,
            )
            rdma_r.start()
            if st < L_HOPS:
                rdma_l = pltpu.make_async_remote_copy(
                    src_ref=comm_l.at[ss],
                    dst_ref=comm_l.at[rs],
                    send_sem=send_l.at[ss],
                    recv_sem=recv_l.at[rs],
                    device_id=(left,),
                    device_id_type=pl.DeviceIdType.MESH,
                )
                rdma_l.start()
            rdma_r.wait()
            if st < R_HOPS - 1:
                cr = _ring2log(lax.rem(r + N_DEV + L_HOPS - st, N_DEV))
                comm_r[rs] = (
                    comm_r[rs].astype(jnp.float32) + p_ref[:, pl.ds(cr * s, s), :]
                ).astype(jnp.bfloat16)
            if st < L_HOPS:
                rdma_l.wait()
                if st < L_HOPS - 1:
                    cl = _ring2log(lax.rem(r + N_DEV - 2 + st, N_DEV))
                    comm_l[rs] = (
                        comm_l[rs].astype(jnp.float32)
                        + p_ref[:, pl.ds(cl * s, s), :]
                    ).astype(jnp.bfloat16)

        out_ref[...] = (
            p_ref[:, pl.ds(my * s, s), :]
            + comm_r[R_HOPS % 2].astype(jnp.float32)
            + comm_l[L_HOPS % 2].astype(jnp.float32)
        )

    return pl.pallas_call(
        body,
        out_shape=jax.ShapeDtypeStruct((B, s, D), partial.dtype),
        in_specs=[pl.BlockSpec(memory_space=pltpu.VMEM)],
        out_specs=pl.BlockSpec(memory_space=pltpu.VMEM),
        scratch_shapes=[
            pltpu.VMEM((2, B, s, D), jnp.bfloat16),
            pltpu.VMEM((2, B, s, D), jnp.bfloat16),
            pltpu.SemaphoreType.DMA((2,)),
            pltpu.SemaphoreType.DMA((2,)),
            pltpu.SemaphoreType.DMA((2,)),
            pltpu.SemaphoreType.DMA((2,)),
        ],
        compiler_params=pltpu.CompilerParams(collective_id=1),
    )(partial)


def _attention(xg, Wq, Wk, Wv, Wo, cos, sin, P):
    B, S, D = xg.shape
    H = Wq.shape[1] // DH
    Wq_h = Wq.reshape(D, H, DH).transpose(1, 0, 2).astype(jnp.bfloat16)
    Wk_h = Wk.reshape(D, H, DH).transpose(1, 0, 2).astype(jnp.bfloat16)
    Wv_h = Wv.reshape(D, H, DH).transpose(1, 0, 2).astype(jnp.bfloat16)
    Wo_h = Wo.reshape(H, DH, D)

    def body(x_ref, wq_ref, wk_ref, wv_ref, wo_ref, cos_ref, sin_ref, p_ref, out_ref):
        h = pl.program_id(1)
        x_b = x_ref[0]
        cos_v = cos_ref[...]
        sin_v = sin_ref[...]
        p_mat = p_ref[...]
        q = jnp.dot(x_b, wq_ref[0], preferred_element_type=jnp.float32)
        k = jnp.dot(x_b, wk_ref[0], preferred_element_type=jnp.float32)
        v = jnp.dot(x_b, wv_ref[0], preferred_element_type=jnp.float32)
        q = q * cos_v + jnp.dot(q, p_mat) * sin_v
        k = k * cos_v + jnp.dot(k, p_mat) * sin_v
        s_ = (
            lax.dot_general(
                q, k, (((1,), (1,)), ((), ())), preferred_element_type=jnp.float32
            )
            * 0.125
        )
        m = jnp.max(s_, axis=1, keepdims=True)
        e = jnp.exp(s_ - m)
        w = e / jnp.sum(e, axis=1, keepdims=True)
        ctx = jnp.dot(w, v, preferred_element_type=jnp.float32)
        contrib = jnp.dot(ctx, wo_ref[0], preferred_element_type=jnp.float32)

        @pl.when(h == 0)
        def _():
            out_ref[0] = contrib

        @pl.when(h != 0)
        def _():
            out_ref[0] = out_ref[0] + contrib

    return pl.pallas_call(
        body,
        grid=(B, H),
        in_specs=[
            pl.BlockSpec((1, S, D), lambda b, h: (b, 0, 0)),
            pl.BlockSpec((1, D, DH), lambda b, h: (h, 0, 0)),
            pl.BlockSpec((1, D, DH), lambda b, h: (h, 0, 0)),
            pl.BlockSpec((1, D, DH), lambda b, h: (h, 0, 0)),
            pl.BlockSpec((1, DH, D), lambda b, h: (h, 0, 0)),
            pl.BlockSpec((S, DH), lambda b, h: (0, 0)),
            pl.BlockSpec((S, DH), lambda b, h: (0, 0)),
            pl.BlockSpec((DH, DH), lambda b, h: (0, 0)),
        ],
        out_specs=pl.BlockSpec((1, S, D), lambda b, h: (b, 0, 0)),
        out_shape=jax.ShapeDtypeStruct((B, S, D), jnp.float32),
    )(xg, Wq_h, Wk_h, Wv_h, Wo_h, cos, sin, P)


def kernel(x, Wq, Wk, Wv, Wo):
    B, s, D = x.shape
    S = N_DEV * s

    xg = _ring_allgather(x.astype(jnp.bfloat16))

    inv = 1.0 / (10000.0 ** (jnp.arange(0, DH, 2, dtype=jnp.float32) / DH))
    pos = jnp.arange(S, dtype=jnp.float32)[:, None] * inv[None, :]
    cos = jnp.repeat(jnp.cos(pos), 2, axis=-1)
    sin = jnp.repeat(jnp.sin(pos), 2, axis=-1)
    row = jnp.arange(DH)[:, None]
    col = jnp.arange(DH)[None, :]
    P = ((col == row + 1) & (row % 2 == 0)).astype(jnp.float32) - (
        (col == row - 1) & (row % 2 == 1)
    ).astype(jnp.float32)

    partial = _attention(xg, Wq, Wk, Wv, Wo, cos, sin, P)
    return _ring_reduce_scatter(partial)
